# baseline (speedup 1.0000x reference)
import functools

import jax
import jax.numpy as jnp
from jax import lax
from jax.experimental import pallas as pl
from jax.experimental.pallas import tpu as pltpu
from jax.experimental.pallas import tpu_sc as plsc

N = 10000
NPAD = 10240
E = 320000
D = 128
NC = 2
NS = 16
NW = NC * NS
EPW = E // NW
CHUNK = 80
NCHUNK = EPW // CHUNK
RPT = NPAD // NS


def _sc_edge_body(src_h, dst_h, et_h, xl_h, xr_h,
                  att_h, num_out, den_out,
                  num_sh,
                  srcb0, dstb0, srcb1, dstb1,
                  xlb, xrb, etb, denb, attb, tbuf,
                  si0, si1, set_, sem1, sem2):
    bufs0 = (srcb0, dstb0)
    bufs1 = (srcb1, dstb1)
    streams = (src_h, dst_h)
    cid = lax.axis_index("c")
    sid = lax.axis_index("s")
    wid = cid * NS + sid

    pltpu.sync_copy(att_h, attb)

    zv = jnp.zeros((16,), jnp.float32)

    def zinit(i, carry):
        for v in range(D // 16):
            xlb[i, pl.ds(v * 16, 16)] = zv
        return carry

    lax.fori_loop(0, CHUNK, zinit, 0)

    def zden(i, carry):
        denb[pl.ds(i * 16, 16)] = zv
        return carry

    lax.fori_loop(0, NPAD // 16, zden, 0)
    for k in range(RPT // CHUNK):
        pltpu.sync_copy(
            xlb, num_sh.at[pl.ds(sid * RPT + k * CHUNK, CHUNK), :])
    plsc.subcore_barrier()

    att_v = [attb[pl.ds(v * 16, 16)] for v in range(8)]
    ii = lax.iota(jnp.int32, 16)

    ebase = wid * EPW

    def issue_streams(c, bufs, si):
        base = ebase + c * CHUNK
        for h, b in zip(streams, bufs):
            pltpu.async_copy(h.at[pl.ds(base, CHUNK)], b, si)

    def wait_streams(c, bufs, si):
        base = ebase + c * CHUNK
        for h, b in zip(streams, bufs):
            pltpu.make_async_copy(h.at[pl.ds(base, CHUNK)], b, si).wait()

    def do_chunk(bufs, c):
        srcb, dstb = bufs
        cp1 = pltpu.async_copy(xl_h.at[srcb], xlb, sem1)
        cp2 = pltpu.async_copy(xr_h.at[dstb], xrb, sem2)
        cp1.wait()
        cp2.wait()
        base = ebase + c * CHUNK
        pltpu.make_async_copy(
            et_h.at[pl.ds(base, CHUNK), :], etb, set_).wait()

        def blk(b, bcarry):
            e0 = b * 16
            dstv = dstb[pl.ds(e0, 16)]
            for i in range(16):
                e = e0 + i
                acc = zv
                for v in range(8):
                    ev = (xlb[e, pl.ds(v * 16, 16)]
                          + xrb[e, pl.ds(v * 16, 16)]
                          + etb[e, pl.ds(v * 16, 16)])
                    ev = jnp.maximum(ev, 0.2 * ev)
                    acc = acc + ev * att_v[v]
                tbuf[i, ...] = acc
            lacc = zv
            for j in range(16):
                lacc = lacc + plsc.load_gather(
                    tbuf, [ii, jnp.full((16,), j, jnp.int32)])
            av = jnp.exp(lacc)
            for i in range(16):
                e = e0 + i
                al = av[i]
                for v in range(8):
                    xlb[e, pl.ds(v * 16, 16)] = (
                        al * xlb[e, pl.ds(v * 16, 16)])
            plsc.addupdate_scatter(denb, [dstv], av)
            return bcarry

        lax.fori_loop(0, CHUNK // 16, blk, 0)
        cn = jnp.minimum(c + 1, NCHUNK - 1)
        pltpu.async_copy(
            et_h.at[pl.ds(ebase + cn * CHUNK, CHUNK), :], etb, set_)
        pltpu.sync_copy(xlb, num_sh.at[dstb], add=True)

    issue_streams(0, bufs0, si0)
    pltpu.async_copy(et_h.at[pl.ds(ebase, CHUNK), :], etb, set_)

    def pair(t, carry):
        c0 = 2 * t
        wait_streams(c0, bufs0, si0)
        issue_streams(c0 + 1, bufs1, si1)
        do_chunk(bufs0, c0)
        wait_streams(c0 + 1, bufs1, si1)
        issue_streams(c0 + 2, bufs0, si0)
        do_chunk(bufs1, c0 + 1)
        return carry

    lax.fori_loop(0, NCHUNK // 2, pair, 0)
    wait_streams(NCHUNK - 1, bufs0, si0)
    do_chunk(bufs0, NCHUNK - 1)
    pltpu.make_async_copy(
        et_h.at[pl.ds(ebase + (NCHUNK - 1) * CHUNK, CHUNK), :],
        etb, set_).wait()

    plsc.subcore_barrier()
    pltpu.sync_copy(num_sh.at[pl.ds(sid * RPT, RPT), :],
                    num_out.at[cid, pl.ds(sid * RPT, RPT), :])
    pltpu.sync_copy(denb, den_out.at[cid, sid, :])


_sc_edge = pl.kernel(
    _sc_edge_body,
    out_type=(jax.ShapeDtypeStruct((NC, NPAD, D), jnp.float32),
              jax.ShapeDtypeStruct((NC, NS, NPAD), jnp.float32)),
    mesh=plsc.VectorSubcoreMesh(core_axis_name="c", subcore_axis_name="s"),
    compiler_params=pltpu.CompilerParams(needs_layout_passes=False),
    scratch_types=[
        pltpu.VMEM_SHARED((NPAD, D), jnp.float32),
        pltpu.VMEM((CHUNK,), jnp.int32),
        pltpu.VMEM((CHUNK,), jnp.int32),
        pltpu.VMEM((CHUNK,), jnp.int32),
        pltpu.VMEM((CHUNK,), jnp.int32),
        pltpu.VMEM((CHUNK, D), jnp.float32),
        pltpu.VMEM((CHUNK, D), jnp.float32),
        pltpu.VMEM((CHUNK, D), jnp.float32),
        pltpu.VMEM((NPAD,), jnp.float32),
        pltpu.VMEM((D,), jnp.float32),
        pltpu.VMEM((16, 16), jnp.float32),
        pltpu.SemaphoreType.DMA,
        pltpu.SemaphoreType.DMA,
        pltpu.SemaphoreType.DMA,
        pltpu.SemaphoreType.DMA,
        pltpu.SemaphoreType.DMA,
    ],
)


def _loopattr_body(dst_h, ea0_h, ea1_h, ea2_h, ea3_h, ls_out,
                   dstb, eab, s0, s1, s2, s3, s4):
    cid = lax.axis_index("c")
    sid = lax.axis_index("s")
    wid = cid * NS + sid

    zv = jnp.zeros((16,), jnp.float32)
    ones = jnp.ones((16,), jnp.float32)
    arrs = [s0, s1, s2, s3, s4]

    def zden(i, carry):
        for a in arrs:
            a[pl.ds(i * 16, 16)] = zv
        return carry

    lax.fori_loop(0, NPAD // 16, zden, 0)

    ebase = wid * EPW

    def chunk(c, carry):
        base = ebase + c * CHUNK
        pltpu.sync_copy(dst_h.at[pl.ds(base, CHUNK)], dstb)
        pltpu.sync_copy(ea0_h.at[pl.ds(base, CHUNK)], eab.at[0])
        pltpu.sync_copy(ea1_h.at[pl.ds(base, CHUNK)], eab.at[1])
        pltpu.sync_copy(ea2_h.at[pl.ds(base, CHUNK)], eab.at[2])
        pltpu.sync_copy(ea3_h.at[pl.ds(base, CHUNK)], eab.at[3])

        def blk(b, bcarry):
            e0 = b * 16
            dstv = dstb[pl.ds(e0, 16)]
            plsc.addupdate_scatter(s0, [dstv], eab[0, pl.ds(e0, 16)])
            plsc.addupdate_scatter(s1, [dstv], eab[1, pl.ds(e0, 16)])
            plsc.addupdate_scatter(s2, [dstv], eab[2, pl.ds(e0, 16)])
            plsc.addupdate_scatter(s3, [dstv], eab[3, pl.ds(e0, 16)])
            plsc.addupdate_scatter(s4, [dstv], ones)
            return bcarry

        lax.fori_loop(0, CHUNK // 16, blk, 0)
        return carry

    lax.fori_loop(0, NCHUNK, chunk, 0)

    for k, a in enumerate(arrs):
        pltpu.sync_copy(a, ls_out.at[cid, sid * 5 + k, :])


_sc_loopattr = pl.kernel(
    _loopattr_body,
    out_type=jax.ShapeDtypeStruct((NC, NS * 5, NPAD), jnp.float32),
    mesh=plsc.VectorSubcoreMesh(core_axis_name="c", subcore_axis_name="s"),
    compiler_params=pltpu.CompilerParams(needs_layout_passes=False),
    scratch_types=[
        pltpu.VMEM((CHUNK,), jnp.int32),
        pltpu.VMEM((4, CHUNK), jnp.float32),
        pltpu.VMEM((NPAD,), jnp.float32),
        pltpu.VMEM((NPAD,), jnp.float32),
        pltpu.VMEM((NPAD,), jnp.float32),
        pltpu.VMEM((NPAD,), jnp.float32),
        pltpu.VMEM((NPAD,), jnp.float32),
    ],
)


def _et_prep_body(a, muc, devc, munc, devnc, wet, o):
    av = a[...]
    a0 = av[:, 0:1]
    a1 = av[:, 1:2]
    d_c = a1 - muc[0, 0]
    d_n = a1 - munc[0, 0]
    t = jnp.where(a0 == 1.0,
                  jnp.exp(-(d_c * d_c) / devc[0, 0]),
                  jnp.exp(-(d_n * d_n) / devnc[0, 0]))
    w = wet[...]
    o[...] = (a0 * w[0:1, :] + t * w[1:2, :]
              + av[:, 2:3] * w[2:3, :] + av[:, 3:4] * w[3:4, :])


def _et_prep(ea, muc, devc, munc, devnc, wet):
    eb = 8000
    grid = (E // eb,)
    aspec = pl.BlockSpec((eb, 4), lambda i: (i, 0))
    sspec = pl.BlockSpec((1, D), lambda i: (0, 0))
    wspec = pl.BlockSpec((4, D), lambda i: (0, 0))
    ospec = pl.BlockSpec((eb, D), lambda i: (i, 0))
    return pl.pallas_call(
        _et_prep_body,
        grid=grid,
        in_specs=[aspec, sspec, sspec, sspec, sspec, wspec],
        out_specs=ospec,
        out_shape=jax.ShapeDtypeStruct((E, D), jnp.float32),
    )(ea, muc, devc, munc, devnc, wet)


def _ea_pack_body(a, muc, devc, munc, devnc, o):
    av = a[...]
    a0 = av[0:1, :]
    a1 = av[1:2, :]
    d_c = a1 - muc[0, 0]
    d_n = a1 - munc[0, 0]
    t = jnp.where(a0 == 1.0,
                  jnp.exp(-(d_c * d_c) / devc[0, 0]),
                  jnp.exp(-(d_n * d_n) / devnc[0, 0]))
    o[0:1, :] = a0
    o[1:2, :] = t
    o[2:4, :] = av[2:4, :]


def _ea_pack(eat, muc, devc, munc, devnc):
    eb = 32000
    grid = (E // eb,)
    aspec = pl.BlockSpec((4, eb), lambda i: (0, i))
    sspec = pl.BlockSpec((1, D), lambda i: (0, 0))
    return pl.pallas_call(
        _ea_pack_body,
        grid=grid,
        in_specs=[aspec, sspec, sspec, sspec, sspec],
        out_specs=aspec,
        out_shape=jax.ShapeDtypeStruct((4, E), jnp.float32),
    )(eat, muc, devc, munc, devnc)


def _in_proj_body(xb, wlt, wrt, bl, br, ol, orr):
    xv = xb[...]
    ol[...] = jnp.dot(xv, wlt[...], preferred_element_type=jnp.float32) + bl[...]
    orr[...] = jnp.dot(xv, wrt[...], preferred_element_type=jnp.float32) + br[...]


def _in_proj(x, wlt, wrt, bl, br):
    blk = 1024
    grid = (NPAD // blk,)
    xspec = pl.BlockSpec((blk, D), lambda i: (i, 0))
    wspec = pl.BlockSpec((D, D), lambda i: (0, 0))
    bspec = pl.BlockSpec((1, D), lambda i: (0, 0))
    return pl.pallas_call(
        _in_proj_body,
        grid=grid,
        in_specs=[xspec, wspec, wspec, bspec, bspec],
        out_specs=[xspec, xspec],
        out_shape=[jax.ShapeDtypeStruct((NPAD, D), jnp.float32)] * 2,
    )(x, wlt, wrt, bl, br)


def _combine_body(first, last, *refs):
    if first:
        (p0, p1, dpart, lsum, xl, xr, wet, att, bias, mscale, wltn, wrtn,
         bln, brn, xlo, xro, lao) = refs
        s = jnp.sum(lsum[...], axis=0)
        degc = jnp.maximum(s[4], 1.0)
        la0 = (s[0] / degc)[:, None]
        la1 = (s[1] / degc)[:, None]
        la2 = (s[2] / degc)[:, None]
        la3 = (s[3] / degc)[:, None]
        lao[:, 0:1] = la0
        lao[:, 1:2] = la1
        lao[:, 2:3] = la2
        lao[:, 3:4] = la3
        lao[:, 4:8] = jnp.zeros_like(lao[:, 4:8])
    elif last:
        (p0, p1, dpart, xl, xr, la8, wet, att, bias, oo) = refs
        la8v = la8[...]
        la0 = la8v[:, 0:1]
        la1 = la8v[:, 1:2]
        la2 = la8v[:, 2:3]
        la3 = la8v[:, 3:4]
    else:
        (p0, p1, dpart, xl, xr, la8, wet, att, bias, mscale, wltn, wrtn,
         bln, brn, xlo, xro) = refs
        la8v = la8[...]
        la0 = la8v[:, 0:1]
        la1 = la8v[:, 1:2]
        la2 = la8v[:, 2:3]
        la3 = la8v[:, 3:4]

    den = jnp.sum(dpart[...], axis=0)[:, None]
    num = p0[...] + p1[...]
    wetv = wet[...]
    lt = (la0 * wetv[0:1, :] + la1 * wetv[1:2, :]
          + la2 * wetv[2:3, :] + la3 * wetv[3:4, :])
    xlv = xl[...]
    es = xlv + xr[...] + lt
    es = jnp.maximum(es, 0.2 * es)
    lsl = jnp.sum(es * att[...], axis=1, keepdims=True)
    asl = jnp.exp(lsl)
    out = (num + asl * xlv) / (den + asl) + bias[...]
    if last:
        oo[...] = out
    else:
        h = jnp.maximum(out, 0.0) * mscale[...]
        xlo[...] = jnp.dot(h, wltn[...], preferred_element_type=jnp.float32) + bln[...]
        xro[...] = jnp.dot(h, wrtn[...], preferred_element_type=jnp.float32) + brn[...]


def _combine(first, last, p0, p1, dpart, lsum, xl, xr, la8, wet, att, bias,
             mscale, wltn, wrtn, bln, brn):
    blk = 1024
    grid = (NPAD // blk,)
    nspec = pl.BlockSpec((blk, D), lambda i: (i, 0))
    dspec = pl.BlockSpec((NW, blk), lambda i: (0, i))
    lsspec = pl.BlockSpec((NW, 5, blk), lambda i: (0, 0, i))
    laspec = pl.BlockSpec((blk, 8), lambda i: (i, 0))
    wetspec = pl.BlockSpec((4, D), lambda i: (0, 0))
    rowspec = pl.BlockSpec((1, D), lambda i: (0, 0))
    wspec = pl.BlockSpec((D, D), lambda i: (0, 0))

    if first:
        in_specs = [nspec, nspec, dspec, lsspec, nspec, nspec, wetspec,
                    rowspec, rowspec, nspec, wspec, wspec, rowspec, rowspec]
        args = (p0, p1, dpart, lsum, xl, xr, wet, att, bias, mscale, wltn,
                wrtn, bln, brn)
        out_specs = [nspec, nspec, laspec]
        out_shape = [jax.ShapeDtypeStruct((NPAD, D), jnp.float32),
                     jax.ShapeDtypeStruct((NPAD, D), jnp.float32),
                     jax.ShapeDtypeStruct((NPAD, 8), jnp.float32)]
    elif last:
        in_specs = [nspec, nspec, dspec, nspec, nspec, laspec,
                    wetspec, rowspec, rowspec]
        args = (p0, p1, dpart, xl, xr, la8, wet, att, bias)
        out_specs = [nspec]
        out_shape = [jax.ShapeDtypeStruct((NPAD, D), jnp.float32)]
    else:
        in_specs = [nspec, nspec, dspec, nspec, nspec, laspec,
                    wetspec, rowspec, rowspec, nspec, wspec, wspec, rowspec,
                    rowspec]
        args = (p0, p1, dpart, xl, xr, la8, wet, att, bias, mscale, wltn,
                wrtn, bln, brn)
        out_specs = [nspec, nspec]
        out_shape = [jax.ShapeDtypeStruct((NPAD, D), jnp.float32),
                     jax.ShapeDtypeStruct((NPAD, D), jnp.float32)]

    outs = pl.pallas_call(
        functools.partial(_combine_body, first, last),
        grid=grid,
        in_specs=in_specs,
        out_specs=out_specs,
        out_shape=out_shape,
    )(*args)
    return outs


def kernel(x, edge_index, edge_attr, params):
    p1, p2, p3 = params['l1'], params['l2'], params['l3']

    def row(v):
        return jnp.full((1, D), v[0], jnp.float32)

    eat = _ea_pack(edge_attr.T,
                   row(params['mu_cov']), row(params['dev_cov']),
                   row(params['mu_ncov']), row(params['dev_ncov']))
    src = edge_index[0]
    dst = edge_index[1]
    ea0, ea1, ea2, ea3 = eat[0], eat[1], eat[2], eat[3]

    msc1 = jnp.where(jax.random.bernoulli(jax.random.key(101), 0.8, (N, D)),
                     jnp.float32(1.25), jnp.float32(0.0))
    msc2 = jnp.where(jax.random.bernoulli(jax.random.key(102), 0.8, (N, D)),
                     jnp.float32(1.25), jnp.float32(0.0))
    pad = ((0, NPAD - N), (0, 0))
    xp = jnp.pad(x, pad)
    msc1 = jnp.pad(msc1, pad)
    msc2 = jnp.pad(msc2, pad)

    def wrow(b):
        return b.reshape(1, D)

    lsum = _sc_loopattr(dst, ea0, ea1, ea2, ea3)
    xl, xr = _in_proj(xp, p1['Wl'].T, p1['Wr'].T, wrow(p1['bl']), wrow(p1['br']))

    wet1, wet2, wet3 = p1['We'].T, p2['We'].T, p3['We'].T
    gp = (row(params['mu_cov']), row(params['dev_cov']),
          row(params['mu_ncov']), row(params['dev_ncov']))
    et1 = _et_prep(edge_attr, *gp, wet1)
    et2 = _et_prep(edge_attr, *gp, wet2)
    et3 = _et_prep(edge_attr, *gp, wet3)

    num, den = _sc_edge(src, dst, et1, xl, xr, p1['att'])
    xl2, xr2, la8 = _combine(True, False, num[0], num[1],
                             den.reshape(NW, NPAD), lsum.reshape(NW, 5, NPAD),
                             xl, xr, None, wet1, wrow(p1['att']),
                             wrow(p1['bias']), msc1, p2['Wl'].T, p2['Wr'].T,
                             wrow(p2['bl']), wrow(p2['br']))

    num, den = _sc_edge(src, dst, et2, xl2, xr2, p2['att'])
    xl3, xr3 = _combine(False, False, num[0], num[1], den.reshape(NW, NPAD),
                        None, xl2, xr2, la8, wet2, wrow(p2['att']),
                        wrow(p2['bias']), msc2, p3['Wl'].T, p3['Wr'].T,
                        wrow(p3['bl']), wrow(p3['br']))

    num, den = _sc_edge(src, dst, et3, xl3, xr3, p3['att'])
    (out,) = _combine(False, True, num[0], num[1], den.reshape(NW, NPAD),
                      None, xl3, xr3, la8, wet3, wrow(p3['att']),
                      wrow(p3['bias']), None, None, None, None, None)
    return out[:N]


# speedup vs baseline: 1.0350x; 1.0350x over previous
import functools

import jax
import jax.numpy as jnp
from jax import lax
from jax.experimental import pallas as pl
from jax.experimental.pallas import tpu as pltpu
from jax.experimental.pallas import tpu_sc as plsc

N = 10000
NPAD = 10240
E = 320000
D = 128
NC = 2
NS = 16
NW = NC * NS
EPW = E // NW
CHUNK = 80
NCHUNK = EPW // CHUNK
RPT = NPAD // NS


def _sc_edge_body(src_h, dst_h, ea0_h, ea1_h, ea2_h, ea3_h, xl_h, xr_h,
                  wet_h, att_h, num_out, den_out,
                  num_sh,
                  srcb0, dstb0, ea0b0, ea1b0, ea2b0, ea3b0,
                  srcb1, dstb1, ea0b1, ea1b1, ea2b1, ea3b1,
                  xlb, xrb, denb, wetb, attb, tbuf,
                  si0, si1, sem1, sem2):
    bufs0 = (srcb0, dstb0, ea0b0, ea1b0, ea2b0, ea3b0)
    bufs1 = (srcb1, dstb1, ea0b1, ea1b1, ea2b1, ea3b1)
    streams = (src_h, dst_h, ea0_h, ea1_h, ea2_h, ea3_h)
    cid = lax.axis_index("c")
    sid = lax.axis_index("s")
    wid = cid * NS + sid

    pltpu.sync_copy(wet_h, wetb)
    pltpu.sync_copy(att_h, attb)

    zv = jnp.zeros((16,), jnp.float32)

    def zinit(i, carry):
        for v in range(D // 16):
            xlb[i, pl.ds(v * 16, 16)] = zv
        return carry

    lax.fori_loop(0, CHUNK, zinit, 0)

    def zden(i, carry):
        denb[pl.ds(i * 16, 16)] = zv
        return carry

    lax.fori_loop(0, NPAD // 16, zden, 0)
    for k in range(RPT // CHUNK):
        pltpu.sync_copy(
            xlb, num_sh.at[pl.ds(sid * RPT + k * CHUNK, CHUNK), :])
    plsc.subcore_barrier()

    wet_v = [[wetb[k, pl.ds(v * 16, 16)] for v in range(8)]
             for k in range(4)]
    att_v = [attb[pl.ds(v * 16, 16)] for v in range(8)]
    ii = lax.iota(jnp.int32, 16)

    ebase = wid * EPW

    def issue_streams(c, bufs, si):
        base = ebase + c * CHUNK
        for h, b in zip(streams, bufs):
            pltpu.async_copy(h.at[pl.ds(base, CHUNK)], b, si)

    def wait_streams(c, bufs, si):
        base = ebase + c * CHUNK
        for h, b in zip(streams, bufs):
            pltpu.make_async_copy(h.at[pl.ds(base, CHUNK)], b, si).wait()

    def do_chunk(bufs):
        srcb, dstb, ea0b, ea1b, ea2b, ea3b = bufs
        cp1 = pltpu.async_copy(xl_h.at[srcb], xlb, sem1)
        cp2 = pltpu.async_copy(xr_h.at[dstb], xrb, sem2)
        cp1.wait()
        cp2.wait()

        def blk(b, bcarry):
            e0 = b * 16
            eav = [ea0b[pl.ds(e0, 16)], ea1b[pl.ds(e0, 16)],
                   ea2b[pl.ds(e0, 16)], ea3b[pl.ds(e0, 16)]]
            dstv = dstb[pl.ds(e0, 16)]
            for i in range(16):
                e = e0 + i
                a0 = eav[0][i]
                a1 = eav[1][i]
                a2 = eav[2][i]
                a3 = eav[3][i]
                acc = zv
                for v in range(8):
                    ev = (xlb[e, pl.ds(v * 16, 16)]
                          + xrb[e, pl.ds(v * 16, 16)]
                          + a0 * wet_v[0][v] + a1 * wet_v[1][v]
                          + a2 * wet_v[2][v] + a3 * wet_v[3][v])
                    ev = jnp.maximum(ev, 0.2 * ev)
                    acc = acc + ev * att_v[v]
                tbuf[i, ...] = acc
            lacc = zv
            for j in range(16):
                lacc = lacc + plsc.load_gather(
                    tbuf, [ii, jnp.full((16,), j, jnp.int32)])
            av = jnp.exp(lacc)
            for i in range(16):
                e = e0 + i
                al = av[i]
                for v in range(8):
                    xlb[e, pl.ds(v * 16, 16)] = (
                        al * xlb[e, pl.ds(v * 16, 16)])
            plsc.addupdate_scatter(denb, [dstv], av)
            return bcarry

        lax.fori_loop(0, CHUNK // 16, blk, 0)
        pltpu.sync_copy(xlb, num_sh.at[dstb], add=True)

    issue_streams(0, bufs0, si0)

    def pair(t, carry):
        c0 = 2 * t
        wait_streams(c0, bufs0, si0)
        issue_streams(c0 + 1, bufs1, si1)
        do_chunk(bufs0)
        wait_streams(c0 + 1, bufs1, si1)
        issue_streams(c0 + 2, bufs0, si0)
        do_chunk(bufs1)
        return carry

    lax.fori_loop(0, NCHUNK // 2, pair, 0)
    wait_streams(NCHUNK - 1, bufs0, si0)
    do_chunk(bufs0)

    plsc.subcore_barrier()
    pltpu.sync_copy(num_sh.at[pl.ds(sid * RPT, RPT), :],
                    num_out.at[cid, pl.ds(sid * RPT, RPT), :])
    pltpu.sync_copy(denb, den_out.at[cid, sid, :])


_sc_edge = pl.kernel(
    _sc_edge_body,
    out_type=(jax.ShapeDtypeStruct((NC, NPAD, D), jnp.float32),
              jax.ShapeDtypeStruct((NC, NS, NPAD), jnp.float32)),
    mesh=plsc.VectorSubcoreMesh(core_axis_name="c", subcore_axis_name="s"),
    compiler_params=pltpu.CompilerParams(needs_layout_passes=False),
    scratch_types=[
        pltpu.VMEM_SHARED((NPAD, D), jnp.float32),
        pltpu.VMEM((CHUNK,), jnp.int32),
        pltpu.VMEM((CHUNK,), jnp.int32),
        pltpu.VMEM((CHUNK,), jnp.float32),
        pltpu.VMEM((CHUNK,), jnp.float32),
        pltpu.VMEM((CHUNK,), jnp.float32),
        pltpu.VMEM((CHUNK,), jnp.float32),
        pltpu.VMEM((CHUNK,), jnp.int32),
        pltpu.VMEM((CHUNK,), jnp.int32),
        pltpu.VMEM((CHUNK,), jnp.float32),
        pltpu.VMEM((CHUNK,), jnp.float32),
        pltpu.VMEM((CHUNK,), jnp.float32),
        pltpu.VMEM((CHUNK,), jnp.float32),
        pltpu.VMEM((CHUNK, D), jnp.float32),
        pltpu.VMEM((CHUNK, D), jnp.float32),
        pltpu.VMEM((NPAD,), jnp.float32),
        pltpu.VMEM((4, D), jnp.float32),
        pltpu.VMEM((D,), jnp.float32),
        pltpu.VMEM((16, 16), jnp.float32),
        pltpu.SemaphoreType.DMA,
        pltpu.SemaphoreType.DMA,
        pltpu.SemaphoreType.DMA,
        pltpu.SemaphoreType.DMA,
    ],
)


def _loopattr_body(dst_h, ea0_h, ea1_h, ea2_h, ea3_h, ls_out,
                   dstb, eab, s0, s1, s2, s3, s4):
    cid = lax.axis_index("c")
    sid = lax.axis_index("s")
    wid = cid * NS + sid

    zv = jnp.zeros((16,), jnp.float32)
    ones = jnp.ones((16,), jnp.float32)
    arrs = [s0, s1, s2, s3, s4]

    def zden(i, carry):
        for a in arrs:
            a[pl.ds(i * 16, 16)] = zv
        return carry

    lax.fori_loop(0, NPAD // 16, zden, 0)

    ebase = wid * EPW

    def chunk(c, carry):
        base = ebase + c * CHUNK
        pltpu.sync_copy(dst_h.at[pl.ds(base, CHUNK)], dstb)
        pltpu.sync_copy(ea0_h.at[pl.ds(base, CHUNK)], eab.at[0])
        pltpu.sync_copy(ea1_h.at[pl.ds(base, CHUNK)], eab.at[1])
        pltpu.sync_copy(ea2_h.at[pl.ds(base, CHUNK)], eab.at[2])
        pltpu.sync_copy(ea3_h.at[pl.ds(base, CHUNK)], eab.at[3])

        def blk(b, bcarry):
            e0 = b * 16
            dstv = dstb[pl.ds(e0, 16)]
            plsc.addupdate_scatter(s0, [dstv], eab[0, pl.ds(e0, 16)])
            plsc.addupdate_scatter(s1, [dstv], eab[1, pl.ds(e0, 16)])
            plsc.addupdate_scatter(s2, [dstv], eab[2, pl.ds(e0, 16)])
            plsc.addupdate_scatter(s3, [dstv], eab[3, pl.ds(e0, 16)])
            plsc.addupdate_scatter(s4, [dstv], ones)
            return bcarry

        lax.fori_loop(0, CHUNK // 16, blk, 0)
        return carry

    lax.fori_loop(0, NCHUNK, chunk, 0)

    for k, a in enumerate(arrs):
        pltpu.sync_copy(a, ls_out.at[cid, sid * 5 + k, :])


_sc_loopattr = pl.kernel(
    _loopattr_body,
    out_type=jax.ShapeDtypeStruct((NC, NS * 5, NPAD), jnp.float32),
    mesh=plsc.VectorSubcoreMesh(core_axis_name="c", subcore_axis_name="s"),
    compiler_params=pltpu.CompilerParams(needs_layout_passes=False),
    scratch_types=[
        pltpu.VMEM((CHUNK,), jnp.int32),
        pltpu.VMEM((4, CHUNK), jnp.float32),
        pltpu.VMEM((NPAD,), jnp.float32),
        pltpu.VMEM((NPAD,), jnp.float32),
        pltpu.VMEM((NPAD,), jnp.float32),
        pltpu.VMEM((NPAD,), jnp.float32),
        pltpu.VMEM((NPAD,), jnp.float32),
    ],
)


def _ea_pack_body(a, muc, devc, munc, devnc, o):
    av = a[...]
    a0 = av[0:1, :]
    a1 = av[1:2, :]
    d_c = a1 - muc[0, 0]
    d_n = a1 - munc[0, 0]
    t = jnp.where(a0 == 1.0,
                  jnp.exp(-(d_c * d_c) / devc[0, 0]),
                  jnp.exp(-(d_n * d_n) / devnc[0, 0]))
    o[0:1, :] = a0
    o[1:2, :] = t
    o[2:4, :] = av[2:4, :]


def _ea_pack(eat, muc, devc, munc, devnc):
    eb = 32000
    grid = (E // eb,)
    aspec = pl.BlockSpec((4, eb), lambda i: (0, i))
    sspec = pl.BlockSpec((1, D), lambda i: (0, 0))
    return pl.pallas_call(
        _ea_pack_body,
        grid=grid,
        in_specs=[aspec, sspec, sspec, sspec, sspec],
        out_specs=aspec,
        out_shape=jax.ShapeDtypeStruct((4, E), jnp.float32),
    )(eat, muc, devc, munc, devnc)


def _in_proj_body(xb, wlt, wrt, bl, br, ol, orr):
    xv = xb[...]
    ol[...] = jnp.dot(xv, wlt[...], preferred_element_type=jnp.float32) + bl[...]
    orr[...] = jnp.dot(xv, wrt[...], preferred_element_type=jnp.float32) + br[...]


def _in_proj(x, wlt, wrt, bl, br):
    blk = 1024
    grid = (NPAD // blk,)
    xspec = pl.BlockSpec((blk, D), lambda i: (i, 0))
    wspec = pl.BlockSpec((D, D), lambda i: (0, 0))
    bspec = pl.BlockSpec((1, D), lambda i: (0, 0))
    return pl.pallas_call(
        _in_proj_body,
        grid=grid,
        in_specs=[xspec, wspec, wspec, bspec, bspec],
        out_specs=[xspec, xspec],
        out_shape=[jax.ShapeDtypeStruct((NPAD, D), jnp.float32)] * 2,
    )(x, wlt, wrt, bl, br)


def _combine_body(first, last, *refs):
    if first:
        (p0, p1, dpart, lsum, xl, xr, wet, att, bias, mscale, wltn, wrtn,
         bln, brn, xlo, xro, lao) = refs
        s = jnp.sum(lsum[...], axis=0)
        degc = jnp.maximum(s[4], 1.0)
        la0 = (s[0] / degc)[:, None]
        la1 = (s[1] / degc)[:, None]
        la2 = (s[2] / degc)[:, None]
        la3 = (s[3] / degc)[:, None]
        lao[:, 0:1] = la0
        lao[:, 1:2] = la1
        lao[:, 2:3] = la2
        lao[:, 3:4] = la3
        lao[:, 4:8] = jnp.zeros_like(lao[:, 4:8])
    elif last:
        (p0, p1, dpart, xl, xr, la8, wet, att, bias, oo) = refs
        la8v = la8[...]
        la0 = la8v[:, 0:1]
        la1 = la8v[:, 1:2]
        la2 = la8v[:, 2:3]
        la3 = la8v[:, 3:4]
    else:
        (p0, p1, dpart, xl, xr, la8, wet, att, bias, mscale, wltn, wrtn,
         bln, brn, xlo, xro) = refs
        la8v = la8[...]
        la0 = la8v[:, 0:1]
        la1 = la8v[:, 1:2]
        la2 = la8v[:, 2:3]
        la3 = la8v[:, 3:4]

    den = jnp.sum(dpart[...], axis=0)[:, None]
    num = p0[...] + p1[...]
    wetv = wet[...]
    lt = (la0 * wetv[0:1, :] + la1 * wetv[1:2, :]
          + la2 * wetv[2:3, :] + la3 * wetv[3:4, :])
    xlv = xl[...]
    es = xlv + xr[...] + lt
    es = jnp.maximum(es, 0.2 * es)
    lsl = jnp.sum(es * att[...], axis=1, keepdims=True)
    asl = jnp.exp(lsl)
    out = (num + asl * xlv) / (den + asl) + bias[...]
    if last:
        oo[...] = out
    else:
        h = jnp.maximum(out, 0.0) * mscale[...]
        xlo[...] = jnp.dot(h, wltn[...], preferred_element_type=jnp.float32) + bln[...]
        xro[...] = jnp.dot(h, wrtn[...], preferred_element_type=jnp.float32) + brn[...]


def _combine(first, last, p0, p1, dpart, lsum, xl, xr, la8, wet, att, bias,
             mscale, wltn, wrtn, bln, brn):
    blk = 1024
    grid = (NPAD // blk,)
    nspec = pl.BlockSpec((blk, D), lambda i: (i, 0))
    dspec = pl.BlockSpec((NW, blk), lambda i: (0, i))
    lsspec = pl.BlockSpec((NW, 5, blk), lambda i: (0, 0, i))
    laspec = pl.BlockSpec((blk, 8), lambda i: (i, 0))
    wetspec = pl.BlockSpec((4, D), lambda i: (0, 0))
    rowspec = pl.BlockSpec((1, D), lambda i: (0, 0))
    wspec = pl.BlockSpec((D, D), lambda i: (0, 0))

    if first:
        in_specs = [nspec, nspec, dspec, lsspec, nspec, nspec, wetspec,
                    rowspec, rowspec, nspec, wspec, wspec, rowspec, rowspec]
        args = (p0, p1, dpart, lsum, xl, xr, wet, att, bias, mscale, wltn,
                wrtn, bln, brn)
        out_specs = [nspec, nspec, laspec]
        out_shape = [jax.ShapeDtypeStruct((NPAD, D), jnp.float32),
                     jax.ShapeDtypeStruct((NPAD, D), jnp.float32),
                     jax.ShapeDtypeStruct((NPAD, 8), jnp.float32)]
    elif last:
        in_specs = [nspec, nspec, dspec, nspec, nspec, laspec,
                    wetspec, rowspec, rowspec]
        args = (p0, p1, dpart, xl, xr, la8, wet, att, bias)
        out_specs = [nspec]
        out_shape = [jax.ShapeDtypeStruct((NPAD, D), jnp.float32)]
    else:
        in_specs = [nspec, nspec, dspec, nspec, nspec, laspec,
                    wetspec, rowspec, rowspec, nspec, wspec, wspec, rowspec,
                    rowspec]
        args = (p0, p1, dpart, xl, xr, la8, wet, att, bias, mscale, wltn,
                wrtn, bln, brn)
        out_specs = [nspec, nspec]
        out_shape = [jax.ShapeDtypeStruct((NPAD, D), jnp.float32),
                     jax.ShapeDtypeStruct((NPAD, D), jnp.float32)]

    outs = pl.pallas_call(
        functools.partial(_combine_body, first, last),
        grid=grid,
        in_specs=in_specs,
        out_specs=out_specs,
        out_shape=out_shape,
    )(*args)
    return outs


def kernel(x, edge_index, edge_attr, params):
    p1, p2, p3 = params['l1'], params['l2'], params['l3']

    def row(v):
        return jnp.full((1, D), v[0], jnp.float32)

    eat = _ea_pack(edge_attr.T,
                   row(params['mu_cov']), row(params['dev_cov']),
                   row(params['mu_ncov']), row(params['dev_ncov']))
    src = edge_index[0]
    dst = edge_index[1]
    ea0, ea1, ea2, ea3 = eat[0], eat[1], eat[2], eat[3]

    msc1 = jnp.where(jax.random.bernoulli(jax.random.key(101), 0.8, (N, D)),
                     jnp.float32(1.25), jnp.float32(0.0))
    msc2 = jnp.where(jax.random.bernoulli(jax.random.key(102), 0.8, (N, D)),
                     jnp.float32(1.25), jnp.float32(0.0))
    pad = ((0, NPAD - N), (0, 0))
    xp = jnp.pad(x, pad)
    msc1 = jnp.pad(msc1, pad)
    msc2 = jnp.pad(msc2, pad)

    def wrow(b):
        return b.reshape(1, D)

    lsum = _sc_loopattr(dst, ea0, ea1, ea2, ea3)
    xl, xr = _in_proj(xp, p1['Wl'].T, p1['Wr'].T, wrow(p1['bl']), wrow(p1['br']))

    wet1, wet2, wet3 = p1['We'].T, p2['We'].T, p3['We'].T

    num, den = _sc_edge(src, dst, ea0, ea1, ea2, ea3, xl, xr, wet1, p1['att'])
    xl2, xr2, la8 = _combine(True, False, num[0], num[1],
                             den.reshape(NW, NPAD), lsum.reshape(NW, 5, NPAD),
                             xl, xr, None, wet1, wrow(p1['att']),
                             wrow(p1['bias']), msc1, p2['Wl'].T, p2['Wr'].T,
                             wrow(p2['bl']), wrow(p2['br']))

    num, den = _sc_edge(src, dst, ea0, ea1, ea2, ea3, xl2, xr2, wet2,
                        p2['att'])
    xl3, xr3 = _combine(False, False, num[0], num[1], den.reshape(NW, NPAD),
                        None, xl2, xr2, la8, wet2, wrow(p2['att']),
                        wrow(p2['bias']), msc2, p3['Wl'].T, p3['Wr'].T,
                        wrow(p3['bl']), wrow(p3['br']))

    num, den = _sc_edge(src, dst, ea0, ea1, ea2, ea3, xl3, xr3, wet3,
                        p3['att'])
    (out,) = _combine(False, True, num[0], num[1], den.reshape(NW, NPAD),
                      None, xl3, xr3, la8, wet3, wrow(p3['att']),
                      wrow(p3['bias']), None, None, None, None, None)
    return out[:N]


# speedup vs baseline: 1.1450x; 1.1063x over previous
import functools

import jax
import jax.numpy as jnp
from jax import lax
from jax.experimental import pallas as pl
from jax.experimental.pallas import tpu as pltpu
from jax.experimental.pallas import tpu_sc as plsc

N = 10000
NPAD = 10240
E = 320000
D = 128
NC = 2
NS = 16
NW = NC * NS
EPW = E // NW
CHUNK = 80
NCHUNK = EPW // CHUNK
RPT = NPAD // NS
CHUNKL = 2000
NCHUNKL = EPW // CHUNKL


def _sc_edge_body(src_h, dst_h, ea0_h, ea1_h, ea2_h, ea3_h, xl_h, xr_h,
                  wet_h, att_h, num_out, den_out,
                  num_sh,
                  srcb0, dstb0, ea0b0, ea1b0, ea2b0, ea3b0,
                  srcb1, dstb1, ea0b1, ea1b1, ea2b1, ea3b1,
                  xlb, xrb, denb, wetb, attb, tbuf,
                  si0, si1, sem1, sem2):
    bufs0 = (srcb0, dstb0, ea0b0, ea1b0, ea2b0, ea3b0)
    bufs1 = (srcb1, dstb1, ea0b1, ea1b1, ea2b1, ea3b1)
    streams = (src_h, dst_h, ea0_h, ea1_h, ea2_h, ea3_h)
    cid = lax.axis_index("c")
    sid = lax.axis_index("s")
    wid = cid * NS + sid

    pltpu.sync_copy(wet_h, wetb)
    pltpu.sync_copy(att_h, attb)

    zv = jnp.zeros((16,), jnp.float32)

    def zinit(i, carry):
        for v in range(D // 16):
            xlb[i, pl.ds(v * 16, 16)] = zv
        return carry

    lax.fori_loop(0, CHUNK, zinit, 0)

    def zden(i, carry):
        denb[pl.ds(i * 16, 16)] = zv
        return carry

    lax.fori_loop(0, NPAD // 16, zden, 0)
    for k in range(RPT // CHUNK):
        pltpu.sync_copy(
            xlb, num_sh.at[pl.ds(sid * RPT + k * CHUNK, CHUNK), :])
    plsc.subcore_barrier()

    wet_v = [[wetb[k, pl.ds(v * 16, 16)] for v in range(8)]
             for k in range(4)]
    att_v = [attb[pl.ds(v * 16, 16)] for v in range(8)]
    ii = lax.iota(jnp.int32, 16)

    ebase = wid * EPW

    def issue_streams(c, bufs, si):
        base = ebase + c * CHUNK
        for h, b in zip(streams, bufs):
            pltpu.async_copy(h.at[pl.ds(base, CHUNK)], b, si)

    def wait_streams(c, bufs, si):
        base = ebase + c * CHUNK
        for h, b in zip(streams, bufs):
            pltpu.make_async_copy(h.at[pl.ds(base, CHUNK)], b, si).wait()

    def do_chunk(bufs):
        srcb, dstb, ea0b, ea1b, ea2b, ea3b = bufs
        cp1 = pltpu.async_copy(xl_h.at[srcb], xlb, sem1)
        cp2 = pltpu.async_copy(xr_h.at[dstb], xrb, sem2)
        cp1.wait()
        cp2.wait()

        def blk(b, bcarry):
            e0 = b * 16
            eav = [ea0b[pl.ds(e0, 16)], ea1b[pl.ds(e0, 16)],
                   ea2b[pl.ds(e0, 16)], ea3b[pl.ds(e0, 16)]]
            dstv = dstb[pl.ds(e0, 16)]
            for i in range(16):
                e = e0 + i
                a0 = eav[0][i]
                a1 = eav[1][i]
                a2 = eav[2][i]
                a3 = eav[3][i]
                acc = zv
                for v in range(8):
                    ev = (xlb[e, pl.ds(v * 16, 16)]
                          + xrb[e, pl.ds(v * 16, 16)]
                          + a0 * wet_v[0][v] + a1 * wet_v[1][v]
                          + a2 * wet_v[2][v] + a3 * wet_v[3][v])
                    ev = jnp.maximum(ev, 0.2 * ev)
                    acc = acc + ev * att_v[v]
                tbuf[i, ...] = acc
            lacc = zv
            for j in range(16):
                lacc = lacc + plsc.load_gather(
                    tbuf, [ii, jnp.full((16,), j, jnp.int32)])
            av = jnp.exp(lacc)
            for i in range(16):
                e = e0 + i
                al = av[i]
                for v in range(8):
                    xlb[e, pl.ds(v * 16, 16)] = (
                        al * xlb[e, pl.ds(v * 16, 16)])
            plsc.addupdate_scatter(denb, [dstv], av)
            return bcarry

        lax.fori_loop(0, CHUNK // 16, blk, 0)
        pltpu.sync_copy(xlb, num_sh.at[dstb], add=True)

    issue_streams(0, bufs0, si0)

    def pair(t, carry):
        c0 = 2 * t
        wait_streams(c0, bufs0, si0)
        issue_streams(c0 + 1, bufs1, si1)
        do_chunk(bufs0)
        wait_streams(c0 + 1, bufs1, si1)
        issue_streams(c0 + 2, bufs0, si0)
        do_chunk(bufs1)
        return carry

    lax.fori_loop(0, NCHUNK // 2, pair, 0)
    wait_streams(NCHUNK - 1, bufs0, si0)
    do_chunk(bufs0)

    plsc.subcore_barrier()
    pltpu.sync_copy(num_sh.at[pl.ds(sid * RPT, RPT), :],
                    num_out.at[cid, pl.ds(sid * RPT, RPT), :])
    pltpu.sync_copy(denb, den_out.at[cid, sid, :])


_sc_edge = pl.kernel(
    _sc_edge_body,
    out_type=(jax.ShapeDtypeStruct((NC, NPAD, D), jnp.float32),
              jax.ShapeDtypeStruct((NC, NS, NPAD), jnp.float32)),
    mesh=plsc.VectorSubcoreMesh(core_axis_name="c", subcore_axis_name="s"),
    compiler_params=pltpu.CompilerParams(needs_layout_passes=False),
    scratch_types=[
        pltpu.VMEM_SHARED((NPAD, D), jnp.float32),
        pltpu.VMEM((CHUNK,), jnp.int32),
        pltpu.VMEM((CHUNK,), jnp.int32),
        pltpu.VMEM((CHUNK,), jnp.float32),
        pltpu.VMEM((CHUNK,), jnp.float32),
        pltpu.VMEM((CHUNK,), jnp.float32),
        pltpu.VMEM((CHUNK,), jnp.float32),
        pltpu.VMEM((CHUNK,), jnp.int32),
        pltpu.VMEM((CHUNK,), jnp.int32),
        pltpu.VMEM((CHUNK,), jnp.float32),
        pltpu.VMEM((CHUNK,), jnp.float32),
        pltpu.VMEM((CHUNK,), jnp.float32),
        pltpu.VMEM((CHUNK,), jnp.float32),
        pltpu.VMEM((CHUNK, D), jnp.float32),
        pltpu.VMEM((CHUNK, D), jnp.float32),
        pltpu.VMEM((NPAD,), jnp.float32),
        pltpu.VMEM((4, D), jnp.float32),
        pltpu.VMEM((D,), jnp.float32),
        pltpu.VMEM((16, 16), jnp.float32),
        pltpu.SemaphoreType.DMA,
        pltpu.SemaphoreType.DMA,
        pltpu.SemaphoreType.DMA,
        pltpu.SemaphoreType.DMA,
    ],
)


def _loopattr_body(dst_h, ea0_h, ea1_h, ea2_h, ea3_h, ls_out,
                   dstb, e0b, e1b, e2b, e3b, s0, s1, s2, s3, s4):
    cid = lax.axis_index("c")
    sid = lax.axis_index("s")
    wid = cid * NS + sid

    zv = jnp.zeros((16,), jnp.float32)
    ones = jnp.ones((16,), jnp.float32)
    arrs = [s0, s1, s2, s3, s4]

    def zden(i, carry):
        for a in arrs:
            a[pl.ds(i * 16, 16)] = zv
        return carry

    lax.fori_loop(0, NPAD // 16, zden, 0)

    ebase = wid * EPW

    def chunk(c, carry):
        base = ebase + c * CHUNKL
        pltpu.sync_copy(dst_h.at[pl.ds(base, CHUNKL)], dstb)
        pltpu.sync_copy(ea0_h.at[pl.ds(base, CHUNKL)], e0b)
        pltpu.sync_copy(ea1_h.at[pl.ds(base, CHUNKL)], e1b)
        pltpu.sync_copy(ea2_h.at[pl.ds(base, CHUNKL)], e2b)
        pltpu.sync_copy(ea3_h.at[pl.ds(base, CHUNKL)], e3b)

        def blk(b, bcarry):
            e0 = b * 16
            dstv = dstb[pl.ds(e0, 16)]
            plsc.addupdate_scatter(s0, [dstv], e0b[pl.ds(e0, 16)])
            plsc.addupdate_scatter(s1, [dstv], e1b[pl.ds(e0, 16)])
            plsc.addupdate_scatter(s2, [dstv], e2b[pl.ds(e0, 16)])
            plsc.addupdate_scatter(s3, [dstv], e3b[pl.ds(e0, 16)])
            plsc.addupdate_scatter(s4, [dstv], ones)
            return bcarry

        lax.fori_loop(0, CHUNKL // 16, blk, 0)
        return carry

    lax.fori_loop(0, NCHUNKL, chunk, 0)

    for k, a in enumerate(arrs):
        pltpu.sync_copy(a, ls_out.at[cid, sid * 5 + k, :])


_sc_loopattr = pl.kernel(
    _loopattr_body,
    out_type=jax.ShapeDtypeStruct((NC, NS * 5, NPAD), jnp.float32),
    mesh=plsc.VectorSubcoreMesh(core_axis_name="c", subcore_axis_name="s"),
    compiler_params=pltpu.CompilerParams(needs_layout_passes=False),
    scratch_types=[
        pltpu.VMEM((CHUNKL,), jnp.int32),
        pltpu.VMEM((CHUNKL,), jnp.float32),
        pltpu.VMEM((CHUNKL,), jnp.float32),
        pltpu.VMEM((CHUNKL,), jnp.float32),
        pltpu.VMEM((CHUNKL,), jnp.float32),
        pltpu.VMEM((NPAD,), jnp.float32),
        pltpu.VMEM((NPAD,), jnp.float32),
        pltpu.VMEM((NPAD,), jnp.float32),
        pltpu.VMEM((NPAD,), jnp.float32),
        pltpu.VMEM((NPAD,), jnp.float32),
    ],
)


def _ea_pack_body(a, muc, devc, munc, devnc, o):
    av = a[...]
    a0 = av[0:1, :]
    a1 = av[1:2, :]
    d_c = a1 - muc[0, 0]
    d_n = a1 - munc[0, 0]
    t = jnp.where(a0 == 1.0,
                  jnp.exp(-(d_c * d_c) / devc[0, 0]),
                  jnp.exp(-(d_n * d_n) / devnc[0, 0]))
    o[0:1, :] = a0
    o[1:2, :] = t
    o[2:4, :] = av[2:4, :]


def _ea_pack(eat, muc, devc, munc, devnc):
    eb = 32000
    grid = (E // eb,)
    aspec = pl.BlockSpec((4, eb), lambda i: (0, i))
    sspec = pl.BlockSpec((1, D), lambda i: (0, 0))
    return pl.pallas_call(
        _ea_pack_body,
        grid=grid,
        in_specs=[aspec, sspec, sspec, sspec, sspec],
        out_specs=aspec,
        out_shape=jax.ShapeDtypeStruct((4, E), jnp.float32),
    )(eat, muc, devc, munc, devnc)


def _in_proj_body(xb, wlt, wrt, bl, br, ol, orr):
    xv = xb[...]
    ol[...] = jnp.dot(xv, wlt[...], preferred_element_type=jnp.float32) + bl[...]
    orr[...] = jnp.dot(xv, wrt[...], preferred_element_type=jnp.float32) + br[...]


def _in_proj(x, wlt, wrt, bl, br):
    blk = 1024
    grid = (NPAD // blk,)
    xspec = pl.BlockSpec((blk, D), lambda i: (i, 0))
    wspec = pl.BlockSpec((D, D), lambda i: (0, 0))
    bspec = pl.BlockSpec((1, D), lambda i: (0, 0))
    return pl.pallas_call(
        _in_proj_body,
        grid=grid,
        in_specs=[xspec, wspec, wspec, bspec, bspec],
        out_specs=[xspec, xspec],
        out_shape=[jax.ShapeDtypeStruct((NPAD, D), jnp.float32)] * 2,
    )(x, wlt, wrt, bl, br)


def _combine_body(first, last, *refs):
    if first:
        (p0, p1, dpart, lsum, xl, xr, wet, att, bias, mscale, wltn, wrtn,
         bln, brn, xlo, xro, lao) = refs
        s = jnp.sum(lsum[...], axis=0)
        degc = jnp.maximum(s[4], 1.0)
        la0 = (s[0] / degc)[:, None]
        la1 = (s[1] / degc)[:, None]
        la2 = (s[2] / degc)[:, None]
        la3 = (s[3] / degc)[:, None]
        lao[:, 0:1] = la0
        lao[:, 1:2] = la1
        lao[:, 2:3] = la2
        lao[:, 3:4] = la3
        lao[:, 4:8] = jnp.zeros_like(lao[:, 4:8])
    elif last:
        (p0, p1, dpart, xl, xr, la8, wet, att, bias, oo) = refs
        la8v = la8[...]
        la0 = la8v[:, 0:1]
        la1 = la8v[:, 1:2]
        la2 = la8v[:, 2:3]
        la3 = la8v[:, 3:4]
    else:
        (p0, p1, dpart, xl, xr, la8, wet, att, bias, mscale, wltn, wrtn,
         bln, brn, xlo, xro) = refs
        la8v = la8[...]
        la0 = la8v[:, 0:1]
        la1 = la8v[:, 1:2]
        la2 = la8v[:, 2:3]
        la3 = la8v[:, 3:4]

    den = jnp.sum(dpart[...], axis=0)[:, None]
    num = p0[...] + p1[...]
    wetv = wet[...]
    lt = (la0 * wetv[0:1, :] + la1 * wetv[1:2, :]
          + la2 * wetv[2:3, :] + la3 * wetv[3:4, :])
    xlv = xl[...]
    es = xlv + xr[...] + lt
    es = jnp.maximum(es, 0.2 * es)
    lsl = jnp.sum(es * att[...], axis=1, keepdims=True)
    asl = jnp.exp(lsl)
    out = (num + asl * xlv) / (den + asl) + bias[...]
    if last:
        oo[...] = out
    else:
        h = jnp.maximum(out, 0.0) * mscale[...]
        xlo[...] = jnp.dot(h, wltn[...], preferred_element_type=jnp.float32) + bln[...]
        xro[...] = jnp.dot(h, wrtn[...], preferred_element_type=jnp.float32) + brn[...]


def _combine(first, last, p0, p1, dpart, lsum, xl, xr, la8, wet, att, bias,
             mscale, wltn, wrtn, bln, brn):
    blk = 1024
    grid = (NPAD // blk,)
    nspec = pl.BlockSpec((blk, D), lambda i: (i, 0))
    dspec = pl.BlockSpec((NW, blk), lambda i: (0, i))
    lsspec = pl.BlockSpec((NW, 5, blk), lambda i: (0, 0, i))
    laspec = pl.BlockSpec((blk, 8), lambda i: (i, 0))
    wetspec = pl.BlockSpec((4, D), lambda i: (0, 0))
    rowspec = pl.BlockSpec((1, D), lambda i: (0, 0))
    wspec = pl.BlockSpec((D, D), lambda i: (0, 0))

    if first:
        in_specs = [nspec, nspec, dspec, lsspec, nspec, nspec, wetspec,
                    rowspec, rowspec, nspec, wspec, wspec, rowspec, rowspec]
        args = (p0, p1, dpart, lsum, xl, xr, wet, att, bias, mscale, wltn,
                wrtn, bln, brn)
        out_specs = [nspec, nspec, laspec]
        out_shape = [jax.ShapeDtypeStruct((NPAD, D), jnp.float32),
                     jax.ShapeDtypeStruct((NPAD, D), jnp.float32),
                     jax.ShapeDtypeStruct((NPAD, 8), jnp.float32)]
    elif last:
        in_specs = [nspec, nspec, dspec, nspec, nspec, laspec,
                    wetspec, rowspec, rowspec]
        args = (p0, p1, dpart, xl, xr, la8, wet, att, bias)
        out_specs = [nspec]
        out_shape = [jax.ShapeDtypeStruct((NPAD, D), jnp.float32)]
    else:
        in_specs = [nspec, nspec, dspec, nspec, nspec, laspec,
                    wetspec, rowspec, rowspec, nspec, wspec, wspec, rowspec,
                    rowspec]
        args = (p0, p1, dpart, xl, xr, la8, wet, att, bias, mscale, wltn,
                wrtn, bln, brn)
        out_specs = [nspec, nspec]
        out_shape = [jax.ShapeDtypeStruct((NPAD, D), jnp.float32),
                     jax.ShapeDtypeStruct((NPAD, D), jnp.float32)]

    outs = pl.pallas_call(
        functools.partial(_combine_body, first, last),
        grid=grid,
        in_specs=in_specs,
        out_specs=out_specs,
        out_shape=out_shape,
    )(*args)
    return outs


def kernel(x, edge_index, edge_attr, params):
    p1, p2, p3 = params['l1'], params['l2'], params['l3']

    def row(v):
        return jnp.full((1, D), v[0], jnp.float32)

    eat = _ea_pack(edge_attr.T,
                   row(params['mu_cov']), row(params['dev_cov']),
                   row(params['mu_ncov']), row(params['dev_ncov']))
    src = edge_index[0]
    dst = edge_index[1]
    ea0, ea1, ea2, ea3 = eat[0], eat[1], eat[2], eat[3]

    msc1 = jnp.where(jax.random.bernoulli(jax.random.key(101), 0.8, (N, D)),
                     jnp.float32(1.25), jnp.float32(0.0))
    msc2 = jnp.where(jax.random.bernoulli(jax.random.key(102), 0.8, (N, D)),
                     jnp.float32(1.25), jnp.float32(0.0))
    pad = ((0, NPAD - N), (0, 0))
    xp = jnp.pad(x, pad)
    msc1 = jnp.pad(msc1, pad)
    msc2 = jnp.pad(msc2, pad)

    def wrow(b):
        return b.reshape(1, D)

    lsum = _sc_loopattr(dst, ea0, ea1, ea2, ea3)
    xl, xr = _in_proj(xp, p1['Wl'].T, p1['Wr'].T, wrow(p1['bl']), wrow(p1['br']))

    wet1, wet2, wet3 = p1['We'].T, p2['We'].T, p3['We'].T

    num, den = _sc_edge(src, dst, ea0, ea1, ea2, ea3, xl, xr, wet1, p1['att'])
    xl2, xr2, la8 = _combine(True, False, num[0], num[1],
                             den.reshape(NW, NPAD), lsum.reshape(NW, 5, NPAD),
                             xl, xr, None, wet1, wrow(p1['att']),
                             wrow(p1['bias']), msc1, p2['Wl'].T, p2['Wr'].T,
                             wrow(p2['bl']), wrow(p2['br']))

    num, den = _sc_edge(src, dst, ea0, ea1, ea2, ea3, xl2, xr2, wet2,
                        p2['att'])
    xl3, xr3 = _combine(False, False, num[0], num[1], den.reshape(NW, NPAD),
                        None, xl2, xr2, la8, wet2, wrow(p2['att']),
                        wrow(p2['bias']), msc2, p3['Wl'].T, p3['Wr'].T,
                        wrow(p3['bl']), wrow(p3['br']))

    num, den = _sc_edge(src, dst, ea0, ea1, ea2, ea3, xl3, xr3, wet3,
                        p3['att'])
    (out,) = _combine(False, True, num[0], num[1], den.reshape(NW, NPAD),
                      None, xl3, xr3, la8, wet3, wrow(p3['att']),
                      wrow(p3['bias']), None, None, None, None, None)
    return out[:N]
